# initial kernel scaffold (unmeasured)
import jax
import jax.numpy as jnp
from jax import lax
from jax.experimental import pallas as pl
from jax.experimental.pallas import tpu as pltpu

N_DEV = 4
N_PANELS = 4


def kernel(x, w_mat):
    m_glob, k_per = x.shape
    _, n_tot = w_mat.shape
    m_per = m_glob // N_DEV
    w_panel = n_tot // N_PANELS

    def body(x_ref, w_ref, out_ref, send_buf, recv_buf,
             amax_send, amax_recv,
             send_sems, recv_sems, amax_send_sems, amax_recv_sems,
             credit_sem):
        my = lax.axis_index("i")
        left = lax.rem(my + N_DEV - 1, N_DEV)
        right = lax.rem(my + 1, N_DEV)

        barrier = pltpu.get_barrier_semaphore()
        for nbr in (left, right):
            pl.semaphore_signal(barrier, inc=1, device_id=(nbr,),
                                device_id_type=pl.DeviceIdType.MESH)
        pl.semaphore_wait(barrier, 2)

        def partial_chunk(c, p):
            x_c = x_ref[pl.ds(c * m_per, m_per), :]
            w_p = w_ref[:, p * w_panel:(p + 1) * w_panel]
            return jnp.dot(x_c, w_p, preferred_element_type=jnp.float32)

        def ring_copy(sem_idx_send, sem_idx_recv, target):
            return pltpu.make_async_remote_copy(
                src_ref=send_buf, dst_ref=recv_buf,
                send_sem=send_sems.at[sem_idx_send],
                recv_sem=recv_sems.at[sem_idx_recv],
                device_id=(target,),
                device_id_type=pl.DeviceIdType.MESH)

        pending = None
        first_send = True
        m_local = jnp.float32(0.0)

        for p in range(N_PANELS):
            c0 = lax.rem(my + N_DEV - 1, N_DEV)
            if pending is not None:
                pending.wait_send()
            send_buf[...] = partial_chunk(c0, p)
            if not first_send:
                pl.semaphore_wait(credit_sem, 1)
            first_send = False
            rdma = ring_copy(p * 3 + 0, p * 3 + 0, right)
            rdma.start()
            pending = rdma

            for h in range(1, N_DEV):
                c = lax.rem(my + 2 * N_DEV - 1 - h, N_DEV)
                ring_copy(p * 3 + h - 1, p * 3 + h - 1, left).wait_recv()
                acc = recv_buf[...]
                if h < N_DEV - 1:
                    pending.wait_send()
                    send_buf[...] = acc + partial_chunk(c, p)
                    pl.semaphore_signal(
                        credit_sem, inc=1, device_id=(left,),
                        device_id_type=pl.DeviceIdType.MESH)
                    pl.semaphore_wait(credit_sem, 1)
                    rdma = ring_copy(p * 3 + h, p * 3 + h, right)
                    rdma.start()
                    pending = rdma
                else:
                    y_p = acc + partial_chunk(c, p)
                    out_ref[:, p * w_panel:(p + 1) * w_panel] = y_p
                    m_local = jnp.maximum(m_local, jnp.max(jnp.abs(y_p)))
                    pl.semaphore_signal(
                        credit_sem, inc=1, device_id=(left,),
                        device_id_type=pl.DeviceIdType.MESH)

        pending.wait_send()

        m = m_local
        for h in range(N_DEV - 1):
            amax_send[...] = jnp.full((8, 128), m, jnp.float32)
            pl.semaphore_wait(credit_sem, 1)
            r = pltpu.make_async_remote_copy(
                src_ref=amax_send, dst_ref=amax_recv,
                send_sem=amax_send_sems.at[h], recv_sem=amax_recv_sems.at[h],
                device_id=(right,), device_id_type=pl.DeviceIdType.MESH)
            r.start()
            pltpu.make_async_remote_copy(
                src_ref=amax_send, dst_ref=amax_recv,
                send_sem=amax_send_sems.at[h], recv_sem=amax_recv_sems.at[h],
                device_id=(left,), device_id_type=pl.DeviceIdType.MESH,
            ).wait_recv()
            m = jnp.maximum(m, amax_recv[0, 0])
            pl.semaphore_signal(credit_sem, inc=1, device_id=(left,),
                                device_id_type=pl.DeviceIdType.MESH)
            r.wait_send()
        pl.semaphore_wait(credit_sem, 1)

        scale = m / 448.0
        inv = 448.0 / m
        for p in range(N_PANELS):
            y_p = out_ref[:, p * w_panel:(p + 1) * w_panel]
            q = jnp.clip(y_p * inv, -448.0, 448.0)
            q = q.astype(jnp.float8_e4m3fn).astype(jnp.float32)
            out_ref[:, p * w_panel:(p + 1) * w_panel] = q * scale

    return pl.pallas_call(
        body,
        out_shape=jax.ShapeDtypeStruct((m_per, n_tot), jnp.float32),
        in_specs=[pl.BlockSpec(memory_space=pltpu.VMEM),
                  pl.BlockSpec(memory_space=pltpu.VMEM)],
        out_specs=pl.BlockSpec(memory_space=pltpu.VMEM),
        scratch_shapes=[
            pltpu.VMEM((m_per, w_panel), jnp.float32),
            pltpu.VMEM((m_per, w_panel), jnp.float32),
            pltpu.VMEM((8, 128), jnp.float32),
            pltpu.VMEM((8, 128), jnp.float32),
            pltpu.SemaphoreType.DMA((3 * N_PANELS,)),
            pltpu.SemaphoreType.DMA((3 * N_PANELS,)),
            pltpu.SemaphoreType.DMA((N_DEV - 1,)),
            pltpu.SemaphoreType.DMA((N_DEV - 1,)),
            pltpu.SemaphoreType.REGULAR,
        ],
        compiler_params=pltpu.CompilerParams(collective_id=0),
    )(x, w_mat)


# baseline (device time: 710882 ns/iter reference)
import jax
import jax.numpy as jnp
from jax import lax
from jax.experimental import pallas as pl
from jax.experimental.pallas import tpu as pltpu

N_DEV = 4
W_PANEL = 512


def kernel(x, w_mat):
    m_glob, k_per = x.shape
    _, n_tot = w_mat.shape
    m_per = m_glob // N_DEV
    half = n_tot // 2
    n_panels = half // W_PANEL
    n_sends = 3 * n_panels

    def body(x_ref, w_hbm, out_hbm, w_vmem, send_bufs, recv_bufs, y_bufs,
             amax_send, amax_recv,
             send_sems, recv_sems, amax_send_sems, amax_recv_sems,
             lsem_w, lsem_y, credit0, credit1, amax_credit):
        my = lax.axis_index("i")
        left = lax.rem(my + N_DEV - 1, N_DEV)
        right = lax.rem(my + 1, N_DEV)

        target = (right, left)
        source = (left, right)
        credit = (credit0, credit1)

        barrier = pltpu.get_barrier_semaphore()
        for nbr in (left, right):
            pl.semaphore_signal(barrier, inc=1, device_id=(nbr,),
                                device_id_type=pl.DeviceIdType.MESH)
        pl.semaphore_wait(barrier, 2)

        def col0(d, p):
            return d * half + p * W_PANEL

        def chunk(d, h):
            if d == 0:
                return lax.rem(my + 2 * N_DEV - 1 - h, N_DEV)
            return lax.rem(my + 1 + h, N_DEV)

        def dot_into(d, s, h):
            x_c = x_ref[pl.ds(chunk(d, h) * m_per, m_per), :]
            send_bufs[d, s, :, :] = jnp.dot(
                x_c, w_vmem[d], preferred_element_type=jnp.float32)

        def ring_copy(d, g, dev):
            return pltpu.make_async_remote_copy(
                src_ref=send_bufs.at[d, g % 2],
                dst_ref=recv_bufs.at[d, g % 2],
                send_sem=send_sems.at[d, g],
                recv_sem=recv_sems.at[d, g],
                device_id=(dev,),
                device_id_type=pl.DeviceIdType.MESH)

        def start_send(d, g):
            if g >= 2:
                pl.semaphore_wait(credit[d], 1)
            rdma = ring_copy(d, g, target[d])
            rdma.start()
            pending[(d, g % 2)] = rdma

        def wait_slot(d, s):
            rdma = pending.pop((d, s), None)
            if rdma is not None:
                rdma.wait_send()

        def consume_credit(d):
            pl.semaphore_signal(credit[d], inc=1, device_id=(source[d],),
                                device_id_type=pl.DeviceIdType.MESH)

        pending = {}
        y_store = [None, None]
        m_local = jnp.float32(0.0)

        for p in range(n_panels):
            wl = []
            for d in (0, 1):
                cp = pltpu.make_async_copy(
                    w_hbm.at[:, pl.ds(col0(d, p), W_PANEL)],
                    w_vmem.at[d], lsem_w.at[d])
                cp.start()
                wl.append(cp)

            for d in (0, 1):
                g = p * 3
                wl[d].wait()
                wait_slot(d, g % 2)
                dot_into(d, g % 2, 0)
                start_send(d, g)

            for h in (1, 2, 3):
                g_send = p * 3 + h
                s = g_send % 2
                for d in (0, 1):
                    wait_slot(d, s)
                    dot_into(d, s, h)
                for d in (0, 1):
                    g_recv = p * 3 + h - 1
                    r = g_recv % 2
                    ring_copy(d, g_recv, source[d]).wait_recv()
                    if h < 3:
                        send_bufs[d, s, :, :] = (
                            send_bufs[d, s, :, :] + recv_bufs[d, r, :, :])
                        consume_credit(d)
                        start_send(d, g_send)
                    else:
                        if y_store[d] is not None:
                            y_store[d].wait()
                        y_val = (send_bufs[d, s, :, :]
                                 + recv_bufs[d, r, :, :])
                        y_bufs[d] = y_val
                        consume_credit(d)
                        m_local = jnp.maximum(
                            m_local, jnp.max(jnp.abs(y_val)))
                        st = pltpu.make_async_copy(
                            y_bufs.at[d],
                            out_hbm.at[:, pl.ds(col0(d, p), W_PANEL)],
                            lsem_y.at[d])
                        st.start()
                        y_store[d] = st

        for d in (0, 1):
            y_store[d].wait()
            wait_slot(d, 0)
            wait_slot(d, 1)

        m = m_local
        for h in range(N_DEV - 1):
            amax_send[...] = jnp.full((8, 128), m, jnp.float32)
            if h >= 1:
                pl.semaphore_wait(amax_credit, 1)
            r = pltpu.make_async_remote_copy(
                src_ref=amax_send, dst_ref=amax_recv,
                send_sem=amax_send_sems.at[h], recv_sem=amax_recv_sems.at[h],
                device_id=(right,), device_id_type=pl.DeviceIdType.MESH)
            r.start()
            pltpu.make_async_remote_copy(
                src_ref=amax_send, dst_ref=amax_recv,
                send_sem=amax_send_sems.at[h], recv_sem=amax_recv_sems.at[h],
                device_id=(left,), device_id_type=pl.DeviceIdType.MESH,
            ).wait_recv()
            m = jnp.maximum(m, amax_recv[0, 0])
            pl.semaphore_signal(amax_credit, inc=1, device_id=(left,),
                                device_id_type=pl.DeviceIdType.MESH)
            r.wait_send()
        pl.semaphore_wait(credit0, 2)
        pl.semaphore_wait(credit1, 2)
        pl.semaphore_wait(amax_credit, 1)

        scale = m / 448.0
        inv = 448.0 / m
        n_tiles = n_tot // W_PANEL
        ep_store = [None, None]
        for t in range(n_tiles):
            b = t % 2
            if ep_store[b] is not None:
                ep_store[b].wait()
            ld = pltpu.make_async_copy(
                out_hbm.at[:, pl.ds(t * W_PANEL, W_PANEL)],
                y_bufs.at[b], lsem_y.at[b])
            ld.start()
            ld.wait()
            q = jnp.clip(y_bufs[b] * inv, -448.0, 448.0)
            q = q.astype(jnp.float8_e4m3fn).astype(jnp.float32)
            y_bufs[b] = q * scale
            st = pltpu.make_async_copy(
                y_bufs.at[b],
                out_hbm.at[:, pl.ds(t * W_PANEL, W_PANEL)], lsem_y.at[b])
            st.start()
            ep_store[b] = st
        for b in (0, 1):
            if ep_store[b] is not None:
                ep_store[b].wait()

    return pl.pallas_call(
        body,
        out_shape=jax.ShapeDtypeStruct((m_per, n_tot), jnp.float32),
        in_specs=[pl.BlockSpec(memory_space=pltpu.MemorySpace.VMEM),
                  pl.BlockSpec(memory_space=pltpu.MemorySpace.HBM)],
        out_specs=pl.BlockSpec(memory_space=pltpu.MemorySpace.HBM),
        scratch_shapes=[
            pltpu.VMEM((2, k_per, W_PANEL), jnp.float32),
            pltpu.VMEM((2, 2, m_per, W_PANEL), jnp.float32),
            pltpu.VMEM((2, 2, m_per, W_PANEL), jnp.float32),
            pltpu.VMEM((2, m_per, W_PANEL), jnp.float32),
            pltpu.VMEM((8, 128), jnp.float32),
            pltpu.VMEM((8, 128), jnp.float32),
            pltpu.SemaphoreType.DMA((2, 3 * half // W_PANEL)),
            pltpu.SemaphoreType.DMA((2, 3 * half // W_PANEL)),
            pltpu.SemaphoreType.DMA((N_DEV - 1,)),
            pltpu.SemaphoreType.DMA((N_DEV - 1,)),
            pltpu.SemaphoreType.DMA((2,)),
            pltpu.SemaphoreType.DMA((2,)),
            pltpu.SemaphoreType.REGULAR,
            pltpu.SemaphoreType.REGULAR,
            pltpu.SemaphoreType.REGULAR,
        ],
        compiler_params=pltpu.CompilerParams(collective_id=0),
    )(x, w_mat)


# device time: 670580 ns/iter; 1.0601x vs baseline; 1.0601x over previous
import jax
import jax.numpy as jnp
from jax import lax
from jax.experimental import pallas as pl
from jax.experimental.pallas import tpu as pltpu

N_DEV = 4
W_PANEL = 512
N_SUB = 2


def kernel(x, w_mat):
    m_glob, k_per = x.shape
    _, n_tot = w_mat.shape
    m_per = m_glob // N_DEV
    half = n_tot // 2
    n_panels = half // W_PANEL
    sub_rows = m_per // N_SUB

    def body(x_ref, w_hbm, out_hbm, w_vmem, send_bufs, recv_bufs, y_bufs,
             amax_send, amax_recv,
             send_sems, recv_sems, amax_send_sems, amax_recv_sems,
             lsem_w, lsem_y, credit0, credit1, amax_credit):
        my = lax.axis_index("i")
        left = lax.rem(my + N_DEV - 1, N_DEV)
        right = lax.rem(my + 1, N_DEV)

        target = (right, left)
        source = (left, right)
        credit = (credit0, credit1)

        barrier = pltpu.get_barrier_semaphore()
        for nbr in (left, right):
            pl.semaphore_signal(barrier, inc=1, device_id=(nbr,),
                                device_id_type=pl.DeviceIdType.MESH)
        pl.semaphore_wait(barrier, 2)

        def col0(d, p):
            return d * half + p * W_PANEL

        def chunk(d, h):
            if d == 0:
                return lax.rem(my + 2 * N_DEV - 1 - h, N_DEV)
            return lax.rem(my + 1 + h, N_DEV)

        def dot_into(d, s, h):
            x_c = x_ref[pl.ds(chunk(d, h) * m_per, m_per), :]
            send_bufs[d, s, :, :] = jnp.dot(
                x_c, w_vmem[d], preferred_element_type=jnp.float32)

        def sub_copy(d, g, sub, dev):
            s = g % 2
            rows = pl.ds(sub * sub_rows, sub_rows)
            return pltpu.make_async_remote_copy(
                src_ref=send_bufs.at[d, s, rows, :],
                dst_ref=recv_bufs.at[d, s, rows, :],
                send_sem=send_sems.at[d, g * N_SUB + sub],
                recv_sem=recv_sems.at[d, g * N_SUB + sub],
                device_id=(dev,),
                device_id_type=pl.DeviceIdType.MESH)

        def start_sub_send(d, g, sub):
            if g >= 2 and sub == 0:
                pl.semaphore_wait(credit[d], 1)
            rdma = sub_copy(d, g, sub, target[d])
            rdma.start()
            pending.setdefault((d, g % 2), []).append(rdma)

        def wait_slot(d, s):
            for rdma in pending.pop((d, s), ()):
                rdma.wait_send()

        def consume_credit(d):
            pl.semaphore_signal(credit[d], inc=1, device_id=(source[d],),
                                device_id_type=pl.DeviceIdType.MESH)

        def add_sub(d, s, sub):
            r0 = sub * sub_rows
            rows = slice(r0, r0 + sub_rows)
            send_bufs[d, s, rows, :] = (
                send_bufs[d, s, rows, :] + recv_bufs[d, s, rows, :])

        pending = {}
        y_store = [None, None]
        m_local = jnp.float32(0.0)

        for p in range(n_panels):
            wl = []
            for d in (0, 1):
                cp = pltpu.make_async_copy(
                    w_hbm.at[:, pl.ds(col0(d, p), W_PANEL)],
                    w_vmem.at[d], lsem_w.at[d])
                cp.start()
                wl.append(cp)

            g0 = p * 3
            for d in (0, 1):
                wl[d].wait()
                wait_slot(d, g0 % 2)
                dot_into(d, g0 % 2, 0)
            for sub in range(N_SUB):
                for d in (0, 1):
                    start_sub_send(d, g0, sub)

            for h in (1, 2, 3):
                g_send = p * 3 + h
                g_recv = g_send - 1
                s = g_send % 2
                r = g_recv % 2
                for d in (0, 1):
                    wait_slot(d, s)
                    dot_into(d, s, h)
                if h < 3:
                    for sub in range(N_SUB):
                        for d in (0, 1):
                            sub_copy(d, g_recv, sub,
                                     source[d]).wait_recv()
                            send_bufs[d, s,
                                      sub * sub_rows:(sub + 1) * sub_rows,
                                      :] = (
                                send_bufs[d, s,
                                          sub * sub_rows:(sub + 1) * sub_rows,
                                          :]
                                + recv_bufs[d, r,
                                            sub * sub_rows:(sub + 1) * sub_rows,
                                            :])
                            start_sub_send(d, g_send, sub)
                    for d in (0, 1):
                        consume_credit(d)
                else:
                    for d in (0, 1):
                        if y_store[d] is not None:
                            y_store[d].wait()
                    for sub in range(N_SUB):
                        for d in (0, 1):
                            sub_copy(d, g_recv, sub,
                                     source[d]).wait_recv()
                            rows = slice(sub * sub_rows,
                                         (sub + 1) * sub_rows)
                            y_val = (send_bufs[d, s, rows, :]
                                     + recv_bufs[d, r, rows, :])
                            y_bufs[d, rows, :] = y_val
                            m_local = jnp.maximum(
                                m_local, jnp.max(jnp.abs(y_val)))
                    for d in (0, 1):
                        consume_credit(d)
                        st = pltpu.make_async_copy(
                            y_bufs.at[d],
                            out_hbm.at[:, pl.ds(col0(d, p), W_PANEL)],
                            lsem_y.at[d])
                        st.start()
                        y_store[d] = st

        for d in (0, 1):
            y_store[d].wait()
            wait_slot(d, 0)
            wait_slot(d, 1)

        m = m_local
        for h in range(N_DEV - 1):
            amax_send[...] = jnp.full((8, 128), m, jnp.float32)
            if h >= 1:
                pl.semaphore_wait(amax_credit, 1)
            r = pltpu.make_async_remote_copy(
                src_ref=amax_send, dst_ref=amax_recv,
                send_sem=amax_send_sems.at[h], recv_sem=amax_recv_sems.at[h],
                device_id=(right,), device_id_type=pl.DeviceIdType.MESH)
            r.start()
            pltpu.make_async_remote_copy(
                src_ref=amax_send, dst_ref=amax_recv,
                send_sem=amax_send_sems.at[h], recv_sem=amax_recv_sems.at[h],
                device_id=(left,), device_id_type=pl.DeviceIdType.MESH,
            ).wait_recv()
            m = jnp.maximum(m, amax_recv[0, 0])
            pl.semaphore_signal(amax_credit, inc=1, device_id=(left,),
                                device_id_type=pl.DeviceIdType.MESH)
            r.wait_send()
        pl.semaphore_wait(credit0, 2)
        pl.semaphore_wait(credit1, 2)
        pl.semaphore_wait(amax_credit, 1)

        scale = m / 448.0
        inv = 448.0 / m
        n_tiles = n_tot // W_PANEL

        def ebuf(i):
            return (y_bufs.at[0], y_bufs.at[1],
                    send_bufs.at[0, 0], send_bufs.at[0, 1])[i]

        def esem(i):
            return (lsem_y.at[0], lsem_y.at[1],
                    lsem_w.at[0], lsem_w.at[1])[i]

        def tile(t):
            return out_hbm.at[:, pl.ds(t * W_PANEL, W_PANEL)]

        loads = {}
        stores = {}
        for t in range(min(4, n_tiles)):
            ld = pltpu.make_async_copy(tile(t), ebuf(t % 4), esem(t % 4))
            ld.start()
            loads[t % 4] = ld
        for t in range(n_tiles):
            b = t % 4
            loads[b].wait()
            q = jnp.clip(ebuf(b)[...] * inv, -448.0, 448.0)
            q = q.astype(jnp.float8_e4m3fn).astype(jnp.float32)
            ebuf(b)[...] = q * scale
            st = pltpu.make_async_copy(ebuf(b), tile(t), esem(b))
            st.start()
            if t + 4 < n_tiles:
                st.wait()
                ld = pltpu.make_async_copy(tile(t + 4), ebuf(b), esem(b))
                ld.start()
                loads[b] = ld
            else:
                stores[b] = st
        for b in stores:
            stores[b].wait()

    return pl.pallas_call(
        body,
        out_shape=jax.ShapeDtypeStruct((m_per, n_tot), jnp.float32),
        in_specs=[pl.BlockSpec(memory_space=pltpu.MemorySpace.VMEM),
                  pl.BlockSpec(memory_space=pltpu.MemorySpace.HBM)],
        out_specs=pl.BlockSpec(memory_space=pltpu.MemorySpace.HBM),
        scratch_shapes=[
            pltpu.VMEM((2, k_per, W_PANEL), jnp.float32),
            pltpu.VMEM((2, 2, m_per, W_PANEL), jnp.float32),
            pltpu.VMEM((2, 2, m_per, W_PANEL), jnp.float32),
            pltpu.VMEM((2, m_per, W_PANEL), jnp.float32),
            pltpu.VMEM((8, 128), jnp.float32),
            pltpu.VMEM((8, 128), jnp.float32),
            pltpu.SemaphoreType.DMA((2, 3 * 8 * N_SUB)),
            pltpu.SemaphoreType.DMA((2, 3 * 8 * N_SUB)),
            pltpu.SemaphoreType.DMA((N_DEV - 1,)),
            pltpu.SemaphoreType.DMA((N_DEV - 1,)),
            pltpu.SemaphoreType.DMA((2,)),
            pltpu.SemaphoreType.DMA((2,)),
            pltpu.SemaphoreType.REGULAR,
            pltpu.SemaphoreType.REGULAR,
            pltpu.SemaphoreType.REGULAR,
        ],
        compiler_params=pltpu.CompilerParams(collective_id=0),
    )(x, w_mat)


# device time: 401840 ns/iter; 1.7691x vs baseline; 1.6688x over previous
import jax
import jax.numpy as jnp
from jax import lax
from jax.experimental import pallas as pl
from jax.experimental.pallas import tpu as pltpu

N_DEV = 4
W_PANEL = 512
N_SUB = 2


def kernel(x, w_mat):
    m_glob, k_per = x.shape
    _, n_tot = w_mat.shape
    m_per = m_glob // N_DEV
    half = n_tot // 2
    n_panels = half // W_PANEL
    sub_rows = m_per // N_SUB

    def body(x_ref, w_hbm, out_hbm, w_vmem, acc_bufs, send_bufs, recv_bufs,
             y_bufs, amax_send, amax_recv,
             send_sems, recv_sems, amax_send_sems, amax_recv_sems,
             lsem_w, lsem_y, credit0, credit1, amax_credit):
        my = lax.axis_index("i")
        left = lax.rem(my + N_DEV - 1, N_DEV)
        right = lax.rem(my + 1, N_DEV)

        target = (right, left)
        source = (left, right)
        credit = (credit0, credit1)

        barrier = pltpu.get_barrier_semaphore()
        for nbr in (left, right):
            pl.semaphore_signal(barrier, inc=1, device_id=(nbr,),
                                device_id_type=pl.DeviceIdType.MESH)
        pl.semaphore_wait(barrier, 2)

        def col0(d, p):
            return d * half + p * W_PANEL

        def chunk(d, h):
            if d == 0:
                return lax.rem(my + 2 * N_DEV - 1 - h, N_DEV)
            return lax.rem(my + 1 + h, N_DEV)

        def partial(d, h):
            x_c = x_ref[pl.ds(chunk(d, h) * m_per, m_per), :]
            return jnp.dot(x_c, w_vmem[d],
                           preferred_element_type=jnp.float32)

        def sub_copy(d, g, sub, dev):
            s = g % 2
            rows = pl.ds(sub * sub_rows, sub_rows)
            return pltpu.make_async_remote_copy(
                src_ref=send_bufs.at[d, s, rows, :],
                dst_ref=recv_bufs.at[d, s, rows, :],
                send_sem=send_sems.at[d, g * N_SUB + sub],
                recv_sem=recv_sems.at[d, g * N_SUB + sub],
                device_id=(dev,),
                device_id_type=pl.DeviceIdType.MESH)

        def start_sub_send(d, g, sub):
            if g >= 2 and sub == 0:
                pl.semaphore_wait(credit[d], 1)
            rdma = sub_copy(d, g, sub, target[d])
            rdma.start()
            pending.setdefault((d, g % 2), []).append(rdma)

        def wait_slot(d, s):
            for rdma in pending.pop((d, s), ()):
                rdma.wait_send()

        def consume_credit(d):
            pl.semaphore_signal(credit[d], inc=1, device_id=(source[d],),
                                device_id_type=pl.DeviceIdType.MESH)

        pending = {}
        y_store = [None, None]
        m_local = jnp.float32(0.0)

        for p in range(n_panels):
            wl = []
            for d in (0, 1):
                cp = pltpu.make_async_copy(
                    w_hbm.at[:, pl.ds(col0(d, p), W_PANEL)],
                    w_vmem.at[d], lsem_w.at[d])
                cp.start()
                wl.append(cp)

            g0 = p * 3
            for d in (0, 1):
                wl[d].wait()
                wait_slot(d, g0 % 2)
                send_bufs[d, g0 % 2, :, :] = partial(d, 0).astype(
                    jnp.bfloat16)
            for sub in range(N_SUB):
                for d in (0, 1):
                    start_sub_send(d, g0, sub)

            for h in (1, 2, 3):
                g_send = p * 3 + h
                g_recv = g_send - 1
                s = g_send % 2
                r = g_recv % 2
                for d in (0, 1):
                    acc_bufs[d, s, :, :] = partial(d, h)
                if h < 3:
                    for d in (0, 1):
                        wait_slot(d, s)
                    for sub in range(N_SUB):
                        rows = slice(sub * sub_rows, (sub + 1) * sub_rows)
                        for d in (0, 1):
                            sub_copy(d, g_recv, sub,
                                     source[d]).wait_recv()
                            send_bufs[d, s, rows, :] = (
                                acc_bufs[d, s, rows, :]
                                + recv_bufs[d, r, rows, :]
                            ).astype(jnp.bfloat16)
                            start_sub_send(d, g_send, sub)
                    for d in (0, 1):
                        consume_credit(d)
                else:
                    for d in (0, 1):
                        if y_store[d] is not None:
                            y_store[d].wait()
                    for sub in range(N_SUB):
                        rows = slice(sub * sub_rows, (sub + 1) * sub_rows)
                        for d in (0, 1):
                            sub_copy(d, g_recv, sub,
                                     source[d]).wait_recv()
                            y_val = (acc_bufs[d, s, rows, :]
                                     + recv_bufs[d, r, rows, :])
                            y_bufs[d, rows, :] = y_val
                            m_local = jnp.maximum(
                                m_local, jnp.max(jnp.abs(y_val)))
                    for d in (0, 1):
                        consume_credit(d)
                        st = pltpu.make_async_copy(
                            y_bufs.at[d],
                            out_hbm.at[:, pl.ds(col0(d, p), W_PANEL)],
                            lsem_y.at[d])
                        st.start()
                        y_store[d] = st

        for d in (0, 1):
            y_store[d].wait()
            wait_slot(d, 0)
            wait_slot(d, 1)

        m = m_local
        for h in range(N_DEV - 1):
            amax_send[...] = jnp.full((8, 128), m, jnp.float32)
            if h >= 1:
                pl.semaphore_wait(amax_credit, 1)
            r = pltpu.make_async_remote_copy(
                src_ref=amax_send, dst_ref=amax_recv,
                send_sem=amax_send_sems.at[h], recv_sem=amax_recv_sems.at[h],
                device_id=(right,), device_id_type=pl.DeviceIdType.MESH)
            r.start()
            pltpu.make_async_remote_copy(
                src_ref=amax_send, dst_ref=amax_recv,
                send_sem=amax_send_sems.at[h], recv_sem=amax_recv_sems.at[h],
                device_id=(left,), device_id_type=pl.DeviceIdType.MESH,
            ).wait_recv()
            m = jnp.maximum(m, amax_recv[0, 0])
            pl.semaphore_signal(amax_credit, inc=1, device_id=(left,),
                                device_id_type=pl.DeviceIdType.MESH)
            r.wait_send()
        pl.semaphore_wait(credit0, 2)
        pl.semaphore_wait(credit1, 2)
        pl.semaphore_wait(amax_credit, 1)

        scale = m / 448.0
        inv = 448.0 / m
        n_tiles = n_tot // W_PANEL

        def ebuf(i):
            return (y_bufs.at[0], y_bufs.at[1],
                    acc_bufs.at[0, 0], acc_bufs.at[0, 1])[i]

        def esem(i):
            return (lsem_y.at[0], lsem_y.at[1],
                    lsem_w.at[0], lsem_w.at[1])[i]

        def tile(t):
            return out_hbm.at[:, pl.ds(t * W_PANEL, W_PANEL)]

        loads = {}
        stores = {}
        for t in range(min(4, n_tiles)):
            ld = pltpu.make_async_copy(tile(t), ebuf(t % 4), esem(t % 4))
            ld.start()
            loads[t % 4] = ld
        for t in range(n_tiles):
            b = t % 4
            loads[b].wait()
            q = jnp.clip(ebuf(b)[...] * inv, -448.0, 448.0)
            q = q.astype(jnp.float8_e4m3fn).astype(jnp.float32)
            ebuf(b)[...] = q * scale
            st = pltpu.make_async_copy(ebuf(b), tile(t), esem(b))
            st.start()
            if t + 4 < n_tiles:
                st.wait()
                ld = pltpu.make_async_copy(tile(t + 4), ebuf(b), esem(b))
                ld.start()
                loads[b] = ld
            else:
                stores[b] = st
        for b in stores:
            stores[b].wait()

    return pl.pallas_call(
        body,
        out_shape=jax.ShapeDtypeStruct((m_per, n_tot), jnp.float32),
        in_specs=[pl.BlockSpec(memory_space=pltpu.MemorySpace.VMEM),
                  pl.BlockSpec(memory_space=pltpu.MemorySpace.HBM)],
        out_specs=pl.BlockSpec(memory_space=pltpu.MemorySpace.HBM),
        scratch_shapes=[
            pltpu.VMEM((2, k_per, W_PANEL), jnp.float32),
            pltpu.VMEM((2, 2, m_per, W_PANEL), jnp.float32),
            pltpu.VMEM((2, 2, m_per, W_PANEL), jnp.bfloat16),
            pltpu.VMEM((2, 2, m_per, W_PANEL), jnp.bfloat16),
            pltpu.VMEM((2, m_per, W_PANEL), jnp.float32),
            pltpu.VMEM((8, 128), jnp.float32),
            pltpu.VMEM((8, 128), jnp.float32),
            pltpu.SemaphoreType.DMA((2, 3 * 8 * N_SUB)),
            pltpu.SemaphoreType.DMA((2, 3 * 8 * N_SUB)),
            pltpu.SemaphoreType.DMA((N_DEV - 1,)),
            pltpu.SemaphoreType.DMA((N_DEV - 1,)),
            pltpu.SemaphoreType.DMA((2,)),
            pltpu.SemaphoreType.DMA((2,)),
            pltpu.SemaphoreType.REGULAR,
            pltpu.SemaphoreType.REGULAR,
            pltpu.SemaphoreType.REGULAR,
        ],
        compiler_params=pltpu.CompilerParams(collective_id=0),
    )(x, w_mat)


# device time: 379745 ns/iter; 1.8720x vs baseline; 1.0582x over previous
import jax
import jax.numpy as jnp
from jax import lax
from jax.experimental import pallas as pl
from jax.experimental.pallas import tpu as pltpu

N_DEV = 4
W_PANEL = 1024
N_SUB = 2


def kernel(x, w_mat):
    m_glob, k_per = x.shape
    _, n_tot = w_mat.shape
    m_per = m_glob // N_DEV
    half = n_tot // 2
    n_panels = half // W_PANEL
    sub_rows = m_per // N_SUB

    def body(x_ref, w_hbm, out_hbm, w_vmem, acc_bufs, send_bufs, recv_bufs,
             y_bufs, amax_send, amax_recv,
             send_sems, recv_sems, amax_send_sems, amax_recv_sems,
             lsem_w, lsem_y, credit0, credit1, amax_credit):
        my = lax.axis_index("i")
        left = lax.rem(my + N_DEV - 1, N_DEV)
        right = lax.rem(my + 1, N_DEV)

        target = (right, left)
        source = (left, right)
        credit = (credit0, credit1)

        barrier = pltpu.get_barrier_semaphore()
        for nbr in (left, right):
            pl.semaphore_signal(barrier, inc=1, device_id=(nbr,),
                                device_id_type=pl.DeviceIdType.MESH)
        pl.semaphore_wait(barrier, 2)

        def col0(d, p):
            return d * half + p * W_PANEL

        def chunk(d, h):
            if d == 0:
                return lax.rem(my + 2 * N_DEV - 1 - h, N_DEV)
            return lax.rem(my + 1 + h, N_DEV)

        def partial(d, h):
            x_c = x_ref[pl.ds(chunk(d, h) * m_per, m_per), :]
            return jnp.dot(x_c, w_vmem[d],
                           preferred_element_type=jnp.float32)

        def sub_copy(d, g, sub, dev):
            s = g % 2
            rows = pl.ds(sub * sub_rows, sub_rows)
            return pltpu.make_async_remote_copy(
                src_ref=send_bufs.at[d, s, rows, :],
                dst_ref=recv_bufs.at[d, s, rows, :],
                send_sem=send_sems.at[d, g * N_SUB + sub],
                recv_sem=recv_sems.at[d, g * N_SUB + sub],
                device_id=(dev,),
                device_id_type=pl.DeviceIdType.MESH)

        def start_sub_send(d, g, sub):
            if g >= 2 and sub == 0:
                pl.semaphore_wait(credit[d], 1)
            rdma = sub_copy(d, g, sub, target[d])
            rdma.start()
            pending.setdefault((d, g % 2), []).append(rdma)

        def wait_slot(d, s):
            for rdma in pending.pop((d, s), ()):
                rdma.wait_send()

        def consume_credit(d):
            pl.semaphore_signal(credit[d], inc=1, device_id=(source[d],),
                                device_id_type=pl.DeviceIdType.MESH)

        pending = {}
        y_store = [None, None]
        m_local = jnp.float32(0.0)

        for p in range(n_panels):
            wl = []
            for d in (0, 1):
                cp = pltpu.make_async_copy(
                    w_hbm.at[:, pl.ds(col0(d, p), W_PANEL)],
                    w_vmem.at[d], lsem_w.at[d])
                cp.start()
                wl.append(cp)

            g0 = p * 3
            for d in (0, 1):
                wl[d].wait()
                wait_slot(d, g0 % 2)
                send_bufs[d, g0 % 2, :, :] = partial(d, 0).astype(
                    jnp.bfloat16)
            for sub in range(N_SUB):
                for d in (0, 1):
                    start_sub_send(d, g0, sub)

            for h in (1, 2, 3):
                g_send = p * 3 + h
                g_recv = g_send - 1
                s = g_send % 2
                r = g_recv % 2
                for d in (0, 1):
                    acc_bufs[d, :, :] = partial(d, h)
                if h < 3:
                    for d in (0, 1):
                        wait_slot(d, s)
                    for sub in range(N_SUB):
                        rows = slice(sub * sub_rows, (sub + 1) * sub_rows)
                        for d in (0, 1):
                            sub_copy(d, g_recv, sub,
                                     source[d]).wait_recv()
                            send_bufs[d, s, rows, :] = (
                                acc_bufs[d, rows, :]
                                + recv_bufs[d, r, rows, :]
                            ).astype(jnp.bfloat16)
                            start_sub_send(d, g_send, sub)
                    for d in (0, 1):
                        consume_credit(d)
                else:
                    for d in (0, 1):
                        if y_store[d] is not None:
                            y_store[d].wait()
                    for sub in range(N_SUB):
                        rows = slice(sub * sub_rows, (sub + 1) * sub_rows)
                        for d in (0, 1):
                            sub_copy(d, g_recv, sub,
                                     source[d]).wait_recv()
                            y_val = (acc_bufs[d, rows, :]
                                     + recv_bufs[d, r, rows, :])
                            y_bufs[d, rows, :] = y_val
                            m_local = jnp.maximum(
                                m_local, jnp.max(jnp.abs(y_val)))
                    for d in (0, 1):
                        consume_credit(d)
                        st = pltpu.make_async_copy(
                            y_bufs.at[d],
                            out_hbm.at[:, pl.ds(col0(d, p), W_PANEL)],
                            lsem_y.at[d])
                        st.start()
                        y_store[d] = st

        for d in (0, 1):
            y_store[d].wait()
            wait_slot(d, 0)
            wait_slot(d, 1)

        m = m_local
        for h in range(N_DEV - 1):
            amax_send[...] = jnp.full((8, 128), m, jnp.float32)
            if h >= 1:
                pl.semaphore_wait(amax_credit, 1)
            r = pltpu.make_async_remote_copy(
                src_ref=amax_send, dst_ref=amax_recv,
                send_sem=amax_send_sems.at[h], recv_sem=amax_recv_sems.at[h],
                device_id=(right,), device_id_type=pl.DeviceIdType.MESH)
            r.start()
            pltpu.make_async_remote_copy(
                src_ref=amax_send, dst_ref=amax_recv,
                send_sem=amax_send_sems.at[h], recv_sem=amax_recv_sems.at[h],
                device_id=(left,), device_id_type=pl.DeviceIdType.MESH,
            ).wait_recv()
            m = jnp.maximum(m, amax_recv[0, 0])
            pl.semaphore_signal(amax_credit, inc=1, device_id=(left,),
                                device_id_type=pl.DeviceIdType.MESH)
            r.wait_send()
        pl.semaphore_wait(credit0, 2)
        pl.semaphore_wait(credit1, 2)
        pl.semaphore_wait(amax_credit, 1)

        scale = m / 448.0
        inv = 448.0 / m
        n_tiles = n_tot // W_PANEL

        def ebuf(i):
            return (y_bufs.at[0], y_bufs.at[1],
                    acc_bufs.at[0], acc_bufs.at[1])[i]

        def esem(i):
            return (lsem_y.at[0], lsem_y.at[1],
                    lsem_w.at[0], lsem_w.at[1])[i]

        def tile(t):
            return out_hbm.at[:, pl.ds(t * W_PANEL, W_PANEL)]

        loads = {}
        stores = {}
        for t in range(min(4, n_tiles)):
            ld = pltpu.make_async_copy(tile(t), ebuf(t % 4), esem(t % 4))
            ld.start()
            loads[t % 4] = ld
        for t in range(n_tiles):
            b = t % 4
            loads[b].wait()
            q = jnp.clip(ebuf(b)[...] * inv, -448.0, 448.0)
            q = q.astype(jnp.float8_e4m3fn).astype(jnp.float32)
            ebuf(b)[...] = q * scale
            st = pltpu.make_async_copy(ebuf(b), tile(t), esem(b))
            st.start()
            if t + 4 < n_tiles:
                st.wait()
                ld = pltpu.make_async_copy(tile(t + 4), ebuf(b), esem(b))
                ld.start()
                loads[b] = ld
            else:
                stores[b] = st
        for b in stores:
            stores[b].wait()

    return pl.pallas_call(
        body,
        out_shape=jax.ShapeDtypeStruct((m_per, n_tot), jnp.float32),
        in_specs=[pl.BlockSpec(memory_space=pltpu.MemorySpace.VMEM),
                  pl.BlockSpec(memory_space=pltpu.MemorySpace.HBM)],
        out_specs=pl.BlockSpec(memory_space=pltpu.MemorySpace.HBM),
        scratch_shapes=[
            pltpu.VMEM((2, k_per, W_PANEL), jnp.float32),
            pltpu.VMEM((2, m_per, W_PANEL), jnp.float32),
            pltpu.VMEM((2, 2, m_per, W_PANEL), jnp.bfloat16),
            pltpu.VMEM((2, 2, m_per, W_PANEL), jnp.bfloat16),
            pltpu.VMEM((2, m_per, W_PANEL), jnp.float32),
            pltpu.VMEM((8, 128), jnp.float32),
            pltpu.VMEM((8, 128), jnp.float32),
            pltpu.SemaphoreType.DMA((2, 3 * 4 * N_SUB)),
            pltpu.SemaphoreType.DMA((2, 3 * 4 * N_SUB)),
            pltpu.SemaphoreType.DMA((N_DEV - 1,)),
            pltpu.SemaphoreType.DMA((N_DEV - 1,)),
            pltpu.SemaphoreType.DMA((2,)),
            pltpu.SemaphoreType.DMA((2,)),
            pltpu.SemaphoreType.REGULAR,
            pltpu.SemaphoreType.REGULAR,
            pltpu.SemaphoreType.REGULAR,
        ],
        compiler_params=pltpu.CompilerParams(
            collective_id=0, vmem_limit_bytes=100 * 1024 * 1024),
    )(x, w_mat)


# device time: 375012 ns/iter; 1.8956x vs baseline; 1.0126x over previous
import jax
import jax.numpy as jnp
from jax import lax
from jax.experimental import pallas as pl
from jax.experimental.pallas import tpu as pltpu

N_DEV = 4
W_PANEL = 1024
N_SUB = 2


def kernel(x, w_mat):
    m_glob, k_per = x.shape
    _, n_tot = w_mat.shape
    m_per = m_glob // N_DEV
    half = n_tot // 2
    n_panels = half // W_PANEL
    sub_rows = m_per // N_SUB

    def body(x_ref, w_hbm, out_hbm, w_vmem, acc_bufs, send_bufs, recv_bufs,
             y_bufs, amax_send, amax_recv,
             send_sems, recv_sems, amax_send_sems, amax_recv_sems,
             lsem_w, lsem_y, credit0, credit1, amax_credit):
        my = lax.axis_index("i")
        left = lax.rem(my + N_DEV - 1, N_DEV)
        right = lax.rem(my + 1, N_DEV)

        target = (right, left)
        source = (left, right)
        credit = (credit0, credit1)

        barrier = pltpu.get_barrier_semaphore()
        for nbr in (left, right):
            pl.semaphore_signal(barrier, inc=1, device_id=(nbr,),
                                device_id_type=pl.DeviceIdType.MESH)
        pl.semaphore_wait(barrier, 2)

        def col0(d, p):
            return d * half + p * W_PANEL

        def chunk(d, h):
            if d == 0:
                return lax.rem(my + 2 * N_DEV - 1 - h, N_DEV)
            return lax.rem(my + 1 + h, N_DEV)

        def partial(d, h):
            x_c = x_ref[pl.ds(chunk(d, h) * m_per, m_per), :]
            return jnp.dot(x_c, w_vmem[d],
                           preferred_element_type=jnp.float32)

        def sub_copy(d, g, sub, dev):
            s = g % 2
            rows = pl.ds(sub * sub_rows, sub_rows)
            return pltpu.make_async_remote_copy(
                src_ref=send_bufs.at[d, s, rows, :],
                dst_ref=recv_bufs.at[d, s, rows, :],
                send_sem=send_sems.at[d, g * N_SUB + sub],
                recv_sem=recv_sems.at[d, g * N_SUB + sub],
                device_id=(dev,),
                device_id_type=pl.DeviceIdType.MESH)

        def start_sub_send(d, g, sub):
            if g >= 2 and sub == 0:
                pl.semaphore_wait(credit[d], 1)
            rdma = sub_copy(d, g, sub, target[d])
            rdma.start()
            pending.setdefault((d, g % 2), []).append(rdma)

        def wait_slot(d, s):
            for rdma in pending.pop((d, s), ()):
                rdma.wait_send()

        def consume_credit(d):
            pl.semaphore_signal(credit[d], inc=1, device_id=(source[d],),
                                device_id_type=pl.DeviceIdType.MESH)

        pending = {}
        y_store = [None, None]
        m_local = jnp.float32(0.0)

        for p in range(n_panels):
            wl = []
            for d in (0, 1):
                cp = pltpu.make_async_copy(
                    w_hbm.at[:, pl.ds(col0(d, p), W_PANEL)],
                    w_vmem.at[d], lsem_w.at[d])
                cp.start()
                wl.append(cp)

            g0 = p * 3
            for d in (0, 1):
                wl[d].wait()
                wait_slot(d, g0 % 2)
                send_bufs[d, g0 % 2, :, :] = partial(d, 0).astype(
                    jnp.bfloat16)
            for sub in range(N_SUB):
                for d in (0, 1):
                    start_sub_send(d, g0, sub)

            for h in (1, 2, 3):
                g_send = p * 3 + h
                g_recv = g_send - 1
                s = g_send % 2
                r = g_recv % 2
                for d in (0, 1):
                    acc_bufs[d, :, :] = partial(d, h)
                if h < 3:
                    for d in (0, 1):
                        wait_slot(d, s)
                    for sub in range(N_SUB):
                        rows = slice(sub * sub_rows, (sub + 1) * sub_rows)
                        for d in (0, 1):
                            sub_copy(d, g_recv, sub,
                                     source[d]).wait_recv()
                            send_bufs[d, s, rows, :] = (
                                acc_bufs[d, rows, :]
                                + recv_bufs[d, r, rows, :]
                            ).astype(jnp.bfloat16)
                            start_sub_send(d, g_send, sub)
                    for d in (0, 1):
                        consume_credit(d)
                else:
                    for d in (0, 1):
                        if y_store[d] is not None:
                            y_store[d].wait()
                    for sub in range(N_SUB):
                        rows = slice(sub * sub_rows, (sub + 1) * sub_rows)
                        for d in (0, 1):
                            sub_copy(d, g_recv, sub,
                                     source[d]).wait_recv()
                            y_val = (acc_bufs[d, rows, :]
                                     + recv_bufs[d, r, rows, :])
                            y_bufs[d, rows, :] = y_val
                            m_local = jnp.maximum(
                                m_local, jnp.max(jnp.abs(y_val)))
                    for d in (0, 1):
                        consume_credit(d)
                        st = pltpu.make_async_copy(
                            y_bufs.at[d],
                            out_hbm.at[:, pl.ds(col0(d, p), W_PANEL)],
                            lsem_y.at[d])
                        st.start()
                        y_store[d] = st

        for d in (0, 1):
            y_store[d].wait()
            wait_slot(d, 0)
            wait_slot(d, 1)

        n_tiles = n_tot // W_PANEL

        def ebuf(i):
            return (y_bufs.at[0], y_bufs.at[1], acc_bufs.at[0],
                    acc_bufs.at[1], w_vmem.at[0], w_vmem.at[1])[i]

        def esem(i):
            return (lsem_y.at[0], lsem_y.at[1], lsem_w.at[0],
                    lsem_w.at[1], amax_send_sems.at[0],
                    amax_recv_sems.at[0])[i]

        def tile(t):
            return out_hbm.at[:, pl.ds(t * W_PANEL, W_PANEL)]

        def start_load(t):
            b = t % 6
            ld = pltpu.make_async_copy(tile(t), ebuf(b), esem(b))
            ld.start()
            loads[b] = ld

        loads = {}
        stores = {}
        for t in range(min(4, n_tiles)):
            start_load(t)

        m = m_local
        for h in range(N_DEV - 1):
            amax_send[...] = jnp.full((8, 128), m, jnp.float32)
            if h >= 1:
                pl.semaphore_wait(amax_credit, 1)
            r = pltpu.make_async_remote_copy(
                src_ref=amax_send, dst_ref=amax_recv,
                send_sem=amax_send_sems.at[h], recv_sem=amax_recv_sems.at[h],
                device_id=(right,), device_id_type=pl.DeviceIdType.MESH)
            r.start()
            pltpu.make_async_remote_copy(
                src_ref=amax_send, dst_ref=amax_recv,
                send_sem=amax_send_sems.at[h], recv_sem=amax_recv_sems.at[h],
                device_id=(left,), device_id_type=pl.DeviceIdType.MESH,
            ).wait_recv()
            m = jnp.maximum(m, amax_recv[0, 0])
            pl.semaphore_signal(amax_credit, inc=1, device_id=(left,),
                                device_id_type=pl.DeviceIdType.MESH)
            r.wait_send()
        pl.semaphore_wait(credit0, 2)
        pl.semaphore_wait(credit1, 2)
        pl.semaphore_wait(amax_credit, 1)

        scale = m / 448.0
        inv = (448.0 / m) * (1.0 - 2.0 ** -20)
        for t in range(4, min(6, n_tiles)):
            start_load(t)
        for t in range(n_tiles):
            b = t % 6
            loads[b].wait()
            q = (ebuf(b)[...] * inv).astype(jnp.float8_e4m3fn)
            ebuf(b)[...] = q.astype(jnp.float32) * scale
            st = pltpu.make_async_copy(ebuf(b), tile(t), esem(b))
            st.start()
            stores[b] = st
            if t + 6 < n_tiles:
                st.wait()
                start_load(t + 6)
        for b in stores:
            stores[b].wait()

    return pl.pallas_call(
        body,
        out_shape=jax.ShapeDtypeStruct((m_per, n_tot), jnp.float32),
        in_specs=[pl.BlockSpec(memory_space=pltpu.MemorySpace.VMEM),
                  pl.BlockSpec(memory_space=pltpu.MemorySpace.HBM)],
        out_specs=pl.BlockSpec(memory_space=pltpu.MemorySpace.HBM),
        scratch_shapes=[
            pltpu.VMEM((2, k_per, W_PANEL), jnp.float32),
            pltpu.VMEM((2, m_per, W_PANEL), jnp.float32),
            pltpu.VMEM((2, 2, m_per, W_PANEL), jnp.bfloat16),
            pltpu.VMEM((2, 2, m_per, W_PANEL), jnp.bfloat16),
            pltpu.VMEM((2, m_per, W_PANEL), jnp.float32),
            pltpu.VMEM((8, 128), jnp.float32),
            pltpu.VMEM((8, 128), jnp.float32),
            pltpu.SemaphoreType.DMA((2, 3 * 4 * N_SUB)),
            pltpu.SemaphoreType.DMA((2, 3 * 4 * N_SUB)),
            pltpu.SemaphoreType.DMA((N_DEV - 1,)),
            pltpu.SemaphoreType.DMA((N_DEV - 1,)),
            pltpu.SemaphoreType.DMA((2,)),
            pltpu.SemaphoreType.DMA((2,)),
            pltpu.SemaphoreType.REGULAR,
            pltpu.SemaphoreType.REGULAR,
            pltpu.SemaphoreType.REGULAR,
        ],
        compiler_params=pltpu.CompilerParams(
            collective_id=0, vmem_limit_bytes=100 * 1024 * 1024),
    )(x, w_mat)


# device time: 347922 ns/iter; 2.0432x vs baseline; 1.0779x over previous
import jax
import jax.numpy as jnp
from jax import lax
from jax.experimental import pallas as pl
from jax.experimental.pallas import tpu as pltpu

N_DEV = 4
W_PANEL = 1024
N_SUB = 2


def kernel(x, w_mat):
    m_glob, k_per = x.shape
    _, n_tot = w_mat.shape
    m_per = m_glob // N_DEV
    half = n_tot // 2
    n_panels = half // W_PANEL
    sub_rows = m_per // N_SUB

    def body(x_ref, w_hbm, out_hbm, w_vmem, acc_bufs, send_bufs, recv_bufs,
             y_bufs, amax_send, amax_recv,
             send_sems, recv_sems, amax_send_sems, amax_recv_sems,
             lsem_w, lsem_y, credit0, credit1, amax_credit):
        my = lax.axis_index("i")
        left = lax.rem(my + N_DEV - 1, N_DEV)
        right = lax.rem(my + 1, N_DEV)

        target = (right, left)
        source = (left, right)
        credit = (credit0, credit1)

        barrier = pltpu.get_barrier_semaphore()
        for nbr in (left, right):
            pl.semaphore_signal(barrier, inc=1, device_id=(nbr,),
                                device_id_type=pl.DeviceIdType.MESH)
        pl.semaphore_wait(barrier, 2)

        def col0(d, p):
            return d * half + p * W_PANEL

        def chunk(d, h):
            if d == 0:
                return lax.rem(my + 2 * N_DEV - 1 - h, N_DEV)
            return lax.rem(my + 1 + h, N_DEV)

        def partial(d, h):
            x_c = x_ref[pl.ds(chunk(d, h) * m_per, m_per), :]
            return jnp.dot(x_c, w_vmem[d],
                           preferred_element_type=jnp.float32)

        def sub_copy(d, g, sub, dev):
            s = g % 2
            rows = pl.ds(sub * sub_rows, sub_rows)
            return pltpu.make_async_remote_copy(
                src_ref=send_bufs.at[d, s, rows, :],
                dst_ref=recv_bufs.at[d, s, rows, :],
                send_sem=send_sems.at[d, g * N_SUB + sub],
                recv_sem=recv_sems.at[d, g * N_SUB + sub],
                device_id=(dev,),
                device_id_type=pl.DeviceIdType.MESH)

        def start_sub_send(d, g, sub):
            if g >= 2 and sub == 0:
                pl.semaphore_wait(credit[d], 1)
            rdma = sub_copy(d, g, sub, target[d])
            rdma.start()
            pending.setdefault((d, g % 2), []).append(rdma)

        def wait_slot(d, s):
            for rdma in pending.pop((d, s), ()):
                rdma.wait_send()

        def consume_credit(d):
            pl.semaphore_signal(credit[d], inc=1, device_id=(source[d],),
                                device_id_type=pl.DeviceIdType.MESH)

        def start_w_loads(p):
            out = []
            for d in (0, 1):
                cp = pltpu.make_async_copy(
                    w_hbm.at[:, pl.ds(col0(d, p), W_PANEL)],
                    w_vmem.at[d], lsem_w.at[d])
                cp.start()
                out.append(cp)
            return out

        def hop0_dots(p):
            g0 = p * 3
            for d in (0, 1):
                wait_slot(d, g0 % 2)
                send_bufs[d, g0 % 2, :, :] = partial(d, 0).astype(
                    jnp.bfloat16)

        pending = {}
        y_store = [None, None]
        m_local = jnp.float32(0.0)

        wl = start_w_loads(0)
        for d in (0, 1):
            wl[d].wait()
        hop0_dots(0)

        for p in range(n_panels):
            g0 = p * 3
            for sub in range(N_SUB):
                for d in (0, 1):
                    start_sub_send(d, g0, sub)

            for h in (1, 2, 3):
                g_send = p * 3 + h
                g_recv = g_send - 1
                s = g_send % 2
                r = g_recv % 2
                for d in (0, 1):
                    acc_bufs[d, :, :] = partial(d, h)
                if h < 3:
                    for d in (0, 1):
                        wait_slot(d, s)
                    for sub in range(N_SUB):
                        rows = slice(sub * sub_rows, (sub + 1) * sub_rows)
                        for d in (0, 1):
                            sub_copy(d, g_recv, sub,
                                     source[d]).wait_recv()
                            send_bufs[d, s, rows, :] = (
                                acc_bufs[d, rows, :]
                                + recv_bufs[d, r, rows, :]
                            ).astype(jnp.bfloat16)
                            start_sub_send(d, g_send, sub)
                    for d in (0, 1):
                        consume_credit(d)
                else:
                    if p + 1 < n_panels:
                        wl = start_w_loads(p + 1)
                        for d in (0, 1):
                            wl[d].wait()
                        hop0_dots(p + 1)
                    for d in (0, 1):
                        if y_store[d] is not None:
                            y_store[d].wait()
                    for sub in range(N_SUB):
                        rows = slice(sub * sub_rows, (sub + 1) * sub_rows)
                        for d in (0, 1):
                            sub_copy(d, g_recv, sub,
                                     source[d]).wait_recv()
                            y_val = (acc_bufs[d, rows, :]
                                     + recv_bufs[d, r, rows, :])
                            y_bufs[d, rows, :] = y_val
                            m_local = jnp.maximum(
                                m_local, jnp.max(jnp.abs(y_val)))
                    for d in (0, 1):
                        consume_credit(d)
                        st = pltpu.make_async_copy(
                            y_bufs.at[d],
                            out_hbm.at[:, pl.ds(col0(d, p), W_PANEL)],
                            lsem_y.at[d])
                        st.start()
                        y_store[d] = st

        for d in (0, 1):
            y_store[d].wait()
            wait_slot(d, 0)
            wait_slot(d, 1)

        n_tiles = n_tot // W_PANEL

        def ebuf(i):
            return (y_bufs.at[0], y_bufs.at[1], acc_bufs.at[0],
                    acc_bufs.at[1], w_vmem.at[0], w_vmem.at[1])[i]

        def esem(i):
            return (lsem_y.at[0], lsem_y.at[1], lsem_w.at[0],
                    lsem_w.at[1], amax_send_sems.at[0],
                    amax_recv_sems.at[0])[i]

        def tile(t):
            return out_hbm.at[:, pl.ds(t * W_PANEL, W_PANEL)]

        def start_load(t):
            b = t % 6
            ld = pltpu.make_async_copy(tile(t), ebuf(b), esem(b))
            ld.start()
            loads[b] = ld

        loads = {}
        stores = {}
        for t in range(min(4, n_tiles)):
            start_load(t)

        m = m_local
        for h in range(N_DEV - 1):
            amax_send[...] = jnp.full((8, 128), m, jnp.float32)
            if h >= 1:
                pl.semaphore_wait(amax_credit, 1)
            r = pltpu.make_async_remote_copy(
                src_ref=amax_send, dst_ref=amax_recv,
                send_sem=amax_send_sems.at[h], recv_sem=amax_recv_sems.at[h],
                device_id=(right,), device_id_type=pl.DeviceIdType.MESH)
            r.start()
            pltpu.make_async_remote_copy(
                src_ref=amax_send, dst_ref=amax_recv,
                send_sem=amax_send_sems.at[h], recv_sem=amax_recv_sems.at[h],
                device_id=(left,), device_id_type=pl.DeviceIdType.MESH,
            ).wait_recv()
            m = jnp.maximum(m, amax_recv[0, 0])
            pl.semaphore_signal(amax_credit, inc=1, device_id=(left,),
                                device_id_type=pl.DeviceIdType.MESH)
            r.wait_send()
        pl.semaphore_wait(credit0, 2)
        pl.semaphore_wait(credit1, 2)
        pl.semaphore_wait(amax_credit, 1)

        scale = m / 448.0
        inv = (448.0 / m) * (1.0 - 2.0 ** -20)
        for t in range(4, min(6, n_tiles)):
            start_load(t)
        for t in range(n_tiles):
            b = t % 6
            loads[b].wait()
            q = (ebuf(b)[...] * inv).astype(jnp.float8_e4m3fn)
            ebuf(b)[...] = q.astype(jnp.float32) * scale
            st = pltpu.make_async_copy(ebuf(b), tile(t), esem(b))
            st.start()
            stores[b] = st
            if t + 6 < n_tiles:
                st.wait()
                start_load(t + 6)
        for b in stores:
            stores[b].wait()

    return pl.pallas_call(
        body,
        out_shape=jax.ShapeDtypeStruct((m_per, n_tot), jnp.float32),
        in_specs=[pl.BlockSpec(memory_space=pltpu.MemorySpace.VMEM),
                  pl.BlockSpec(memory_space=pltpu.MemorySpace.HBM)],
        out_specs=pl.BlockSpec(memory_space=pltpu.MemorySpace.HBM),
        scratch_shapes=[
            pltpu.VMEM((2, k_per, W_PANEL), jnp.float32),
            pltpu.VMEM((2, m_per, W_PANEL), jnp.float32),
            pltpu.VMEM((2, 2, m_per, W_PANEL), jnp.bfloat16),
            pltpu.VMEM((2, 2, m_per, W_PANEL), jnp.bfloat16),
            pltpu.VMEM((2, m_per, W_PANEL), jnp.float32),
            pltpu.VMEM((8, 128), jnp.float32),
            pltpu.VMEM((8, 128), jnp.float32),
            pltpu.SemaphoreType.DMA((2, 3 * 4 * N_SUB)),
            pltpu.SemaphoreType.DMA((2, 3 * 4 * N_SUB)),
            pltpu.SemaphoreType.DMA((N_DEV - 1,)),
            pltpu.SemaphoreType.DMA((N_DEV - 1,)),
            pltpu.SemaphoreType.DMA((2,)),
            pltpu.SemaphoreType.DMA((2,)),
            pltpu.SemaphoreType.REGULAR,
            pltpu.SemaphoreType.REGULAR,
            pltpu.SemaphoreType.REGULAR,
        ],
        compiler_params=pltpu.CompilerParams(
            collective_id=0, vmem_limit_bytes=100 * 1024 * 1024),
    )(x, w_mat)


# device time: 339893 ns/iter; 2.0915x vs baseline; 1.0236x over previous
import jax
import jax.numpy as jnp
from jax import lax
from jax.experimental import pallas as pl
from jax.experimental.pallas import tpu as pltpu

N_DEV = 4
W_PANEL = 1024
N_SUB = 2


def kernel(x, w_mat):
    m_glob, k_per = x.shape
    _, n_tot = w_mat.shape
    m_per = m_glob // N_DEV
    half = n_tot // 2
    n_panels = half // W_PANEL
    sub_rows = m_per // N_SUB

    def body(x_ref, w_hbm, out_hbm, w_vmem, acc_bufs, send_bufs, recv_bufs,
             y_bufs, amax_send, amax_recv,
             send_sems, recv_sems, amax_send_sems, amax_recv_sems,
             lsem_w, lsem_y, credit0, credit1, amax_credit):
        my = lax.axis_index("i")
        left = lax.rem(my + N_DEV - 1, N_DEV)
        right = lax.rem(my + 1, N_DEV)

        target = (right, left)
        source = (left, right)
        credit = (credit0, credit1)

        barrier = pltpu.get_barrier_semaphore()
        for nbr in (left, right):
            pl.semaphore_signal(barrier, inc=1, device_id=(nbr,),
                                device_id_type=pl.DeviceIdType.MESH)
        pl.semaphore_wait(barrier, 2)

        def col0(d, p):
            return d * half + p * W_PANEL

        def chunk(d, h):
            if d == 0:
                return lax.rem(my + 2 * N_DEV - 1 - h, N_DEV)
            return lax.rem(my + 1 + h, N_DEV)

        def partial(d, h):
            x_c = x_ref[pl.ds(chunk(d, h) * m_per, m_per), :]
            return jnp.dot(x_c, w_vmem[d],
                           preferred_element_type=jnp.float32)

        def sub_copy(d, g, sub, dev):
            s = g % 2
            rows = pl.ds(sub * sub_rows, sub_rows)
            return pltpu.make_async_remote_copy(
                src_ref=send_bufs.at[d, s, rows, :],
                dst_ref=recv_bufs.at[d, s, rows, :],
                send_sem=send_sems.at[d, g * N_SUB + sub],
                recv_sem=recv_sems.at[d, g * N_SUB + sub],
                device_id=(dev,),
                device_id_type=pl.DeviceIdType.MESH)

        def start_sub_send(d, g, sub):
            if g >= 2 and sub == 0:
                pl.semaphore_wait(credit[d], 1)
            rdma = sub_copy(d, g, sub, target[d])
            rdma.start()
            pending.setdefault((d, g % 2), []).append(rdma)

        def wait_slot(d, s):
            for rdma in pending.pop((d, s), ()):
                rdma.wait_send()

        def consume_credit(d):
            pl.semaphore_signal(credit[d], inc=1, device_id=(source[d],),
                                device_id_type=pl.DeviceIdType.MESH)

        def start_w_loads(p):
            out = []
            for d in (0, 1):
                cp = pltpu.make_async_copy(
                    w_hbm.at[:, pl.ds(col0(d, p), W_PANEL)],
                    w_vmem.at[d], lsem_w.at[d])
                cp.start()
                out.append(cp)
            return out

        def hop0_dots(p):
            g0 = p * 3
            for d in (0, 1):
                wait_slot(d, g0 % 2)
                send_bufs[d, g0 % 2, :, :] = partial(d, 0).astype(
                    jnp.bfloat16)

        pending = {}
        y_store = [None, None]
        m_local = jnp.float32(0.0)

        wl = start_w_loads(0)
        for d in (0, 1):
            wl[d].wait()
        hop0_dots(0)

        for p in range(n_panels):
            if p == 0:
                for sub in range(N_SUB):
                    for d in (0, 1):
                        start_sub_send(d, 0, sub)

            for h in (1, 2, 3):
                g_send = p * 3 + h
                g_recv = g_send - 1
                s = g_send % 2
                r = g_recv % 2
                for d in (0, 1):
                    acc_bufs[d, :, :] = partial(d, h)
                if h < 3:
                    for d in (0, 1):
                        wait_slot(d, s)
                    for sub in range(N_SUB):
                        rows = slice(sub * sub_rows, (sub + 1) * sub_rows)
                        for d in (0, 1):
                            sub_copy(d, g_recv, sub,
                                     source[d]).wait_recv()
                            send_bufs[d, s, rows, :] = (
                                acc_bufs[d, rows, :]
                                + recv_bufs[d, r, rows, :]
                            ).astype(jnp.bfloat16)
                            start_sub_send(d, g_send, sub)
                    for d in (0, 1):
                        consume_credit(d)
                else:
                    if p + 1 < n_panels:
                        wl = start_w_loads(p + 1)
                        for d in (0, 1):
                            wl[d].wait()
                        hop0_dots(p + 1)
                        for sub in range(N_SUB):
                            for d in (0, 1):
                                start_sub_send(d, (p + 1) * 3, sub)
                    for d in (0, 1):
                        if y_store[d] is not None:
                            y_store[d].wait()
                    for sub in range(N_SUB):
                        rows = slice(sub * sub_rows, (sub + 1) * sub_rows)
                        for d in (0, 1):
                            sub_copy(d, g_recv, sub,
                                     source[d]).wait_recv()
                            y_val = (acc_bufs[d, rows, :]
                                     + recv_bufs[d, r, rows, :])
                            y_bufs[d, rows, :] = y_val
                            m_local = jnp.maximum(
                                m_local, jnp.max(jnp.abs(y_val)))
                    for d in (0, 1):
                        consume_credit(d)
                        st = pltpu.make_async_copy(
                            y_bufs.at[d],
                            out_hbm.at[:, pl.ds(col0(d, p), W_PANEL)],
                            lsem_y.at[d])
                        st.start()
                        y_store[d] = st

        for d in (0, 1):
            y_store[d].wait()
            wait_slot(d, 0)
            wait_slot(d, 1)

        n_tiles = n_tot // W_PANEL

        def ebuf(i):
            return (y_bufs.at[0], y_bufs.at[1], acc_bufs.at[0],
                    acc_bufs.at[1], w_vmem.at[0], w_vmem.at[1])[i]

        def esem(i):
            return (lsem_y.at[0], lsem_y.at[1], lsem_w.at[0],
                    lsem_w.at[1], amax_send_sems.at[0],
                    amax_recv_sems.at[0])[i]

        def tile(t):
            return out_hbm.at[:, pl.ds(t * W_PANEL, W_PANEL)]

        def start_load(t):
            b = t % 6
            ld = pltpu.make_async_copy(tile(t), ebuf(b), esem(b))
            ld.start()
            loads[b] = ld

        loads = {}
        stores = {}
        for t in range(min(4, n_tiles)):
            start_load(t)

        m = m_local
        for h in range(N_DEV - 1):
            amax_send[...] = jnp.full((8, 128), m, jnp.float32)
            if h >= 1:
                pl.semaphore_wait(amax_credit, 1)
            r = pltpu.make_async_remote_copy(
                src_ref=amax_send, dst_ref=amax_recv,
                send_sem=amax_send_sems.at[h], recv_sem=amax_recv_sems.at[h],
                device_id=(right,), device_id_type=pl.DeviceIdType.MESH)
            r.start()
            pltpu.make_async_remote_copy(
                src_ref=amax_send, dst_ref=amax_recv,
                send_sem=amax_send_sems.at[h], recv_sem=amax_recv_sems.at[h],
                device_id=(left,), device_id_type=pl.DeviceIdType.MESH,
            ).wait_recv()
            m = jnp.maximum(m, amax_recv[0, 0])
            pl.semaphore_signal(amax_credit, inc=1, device_id=(left,),
                                device_id_type=pl.DeviceIdType.MESH)
            r.wait_send()
        pl.semaphore_wait(credit0, 2)
        pl.semaphore_wait(credit1, 2)
        pl.semaphore_wait(amax_credit, 1)

        scale = m / 448.0
        inv = (448.0 / m) * (1.0 - 2.0 ** -20)
        for t in range(4, min(6, n_tiles)):
            start_load(t)
        for t in range(n_tiles):
            b = t % 6
            loads[b].wait()
            q = (ebuf(b)[...] * inv).astype(jnp.float8_e4m3fn)
            ebuf(b)[...] = q.astype(jnp.float32) * scale
            st = pltpu.make_async_copy(ebuf(b), tile(t), esem(b))
            st.start()
            stores[b] = st
            if t + 6 < n_tiles:
                st.wait()
                start_load(t + 6)
        for b in stores:
            stores[b].wait()

    return pl.pallas_call(
        body,
        out_shape=jax.ShapeDtypeStruct((m_per, n_tot), jnp.float32),
        in_specs=[pl.BlockSpec(memory_space=pltpu.MemorySpace.VMEM),
                  pl.BlockSpec(memory_space=pltpu.MemorySpace.HBM)],
        out_specs=pl.BlockSpec(memory_space=pltpu.MemorySpace.HBM),
        scratch_shapes=[
            pltpu.VMEM((2, k_per, W_PANEL), jnp.float32),
            pltpu.VMEM((2, m_per, W_PANEL), jnp.float32),
            pltpu.VMEM((2, 2, m_per, W_PANEL), jnp.bfloat16),
            pltpu.VMEM((2, 2, m_per, W_PANEL), jnp.bfloat16),
            pltpu.VMEM((2, m_per, W_PANEL), jnp.float32),
            pltpu.VMEM((8, 128), jnp.float32),
            pltpu.VMEM((8, 128), jnp.float32),
            pltpu.SemaphoreType.DMA((2, 3 * 4 * N_SUB)),
            pltpu.SemaphoreType.DMA((2, 3 * 4 * N_SUB)),
            pltpu.SemaphoreType.DMA((N_DEV - 1,)),
            pltpu.SemaphoreType.DMA((N_DEV - 1,)),
            pltpu.SemaphoreType.DMA((2,)),
            pltpu.SemaphoreType.DMA((2,)),
            pltpu.SemaphoreType.REGULAR,
            pltpu.SemaphoreType.REGULAR,
            pltpu.SemaphoreType.REGULAR,
        ],
        compiler_params=pltpu.CompilerParams(
            collective_id=0, vmem_limit_bytes=100 * 1024 * 1024),
    )(x, w_mat)
